# baseline (speedup 1.0000x reference)
import math

import jax
import jax.numpy as jnp
from jax.experimental import pallas as pl
from jax.experimental.pallas import tpu as pltpu

SEQ = 2048
IN_DIM = 1024
P = 256
H = 8
HD = 32
NPENT = 1024
KW = 128
QT = 256
NT = SEQ // QT
MARGIN = 192
BW = QT + 2 * MARGIN
NW = SEQ - KW + 1
WS = 384
SCALE = 1.0 / math.sqrt(HD)


def _gelu(x):
    return 0.5 * x * (1.0 + jax.lax.erf(x * (1.0 / math.sqrt(2.0))))


def _ln(x, g, b):
    mu = jnp.mean(x, axis=-1, keepdims=True)
    xc = x - mu
    var = jnp.mean(xc * xc, axis=-1, keepdims=True)
    return xc / jnp.sqrt(var + 1e-5) * g + b


def _megakernel(x_ref, pentT_ref, pos_ref,
                w_in_ref, b_in_ref, g_in_ref, bt_in_ref,
                w_q_ref, b_q_ref, w_k_ref, b_k_ref, w_v_ref, b_v_ref,
                w_out_ref, b_out_ref,
                wa0_ref, ba0_ref, g0_ref, bb0_ref, wb0_ref, bo0_ref,
                wa1_ref, ba1_ref, g1_ref, bb1_ref, wb1_ref, bo1_ref,
                wa2_ref, ba2_ref, g2_ref, bb2_ref, wb2_ref, bo2_ref,
                out0_ref, out1_ref, out2_ref,
                qs_ref, ksp_ref, vsp_ref, psp_ref, rcp_ref,
                psc_ref, psr_ref, hir_ref, acc_ref):
    pid = pl.program_id(0)

    @pl.when(pid == 0)
    def _preamble():
        centT = (pentT_ref[0] + pentT_ref[1] + pentT_ref[2]
                 + pentT_ref[3] + pentT_ref[4]) / 5.0
        cn = jnp.sqrt(jnp.sum(centT * centT, axis=0, keepdims=True))
        centT = centT / jnp.maximum(cn, 1e-12)

        pr = jnp.dot(x_ref[...], w_in_ref[...],
                     preferred_element_type=jnp.float32) + b_in_ref[...]
        proj = _gelu(_ln(pr, g_in_ref[...], bt_in_ref[...]))

        n = jnp.sqrt(jnp.sum(proj * proj, axis=-1, keepdims=True))
        fn = proj / jnp.maximum(n, 1e-12)
        sims = jnp.dot(fn, centT, preferred_element_type=jnp.float32)
        rowmax = jnp.max(sims, axis=-1, keepdims=True)
        lane_a = jax.lax.broadcasted_iota(jnp.int32, (SEQ, NPENT), 1)
        aidx = jnp.min(jnp.where(sims == rowmax, lane_a, NPENT),
                       axis=-1, keepdims=True)
        onehot = (lane_a == aidx).astype(jnp.float32)
        pos_col = jnp.dot(onehot, jnp.transpose(pos_ref[...]),
                          preferred_element_type=jnp.float32)
        pos_row = jnp.transpose(pos_col)

        iota_cr = jax.lax.broadcasted_iota(jnp.int32, (QT, SEQ), 0)
        iota_cc = jax.lax.broadcasted_iota(jnp.int32, (QT, SEQ), 1)
        rcount_row = jnp.zeros((1, SEQ), jnp.int32)
        cntlt_row = jnp.zeros((1, SEQ), jnp.int32)
        for c0 in range(0, SEQ, QT):
            pc = pos_col[c0:c0 + QT, :]
            jlt = (iota_cr + c0) < iota_cc
            rcount_row = rcount_row + jnp.sum(
                ((pc == pos_row) & jlt).astype(jnp.int32),
                axis=0, keepdims=True)
            cntlt_row = cntlt_row + jnp.sum(
                (pc < pos_row).astype(jnp.int32), axis=0, keepdims=True)
        rank_row = cntlt_row + rcount_row

        ext = jnp.concatenate(
            [pos_col, jnp.transpose(rcount_row).astype(jnp.float32)],
            axis=1)
        projb = proj.astype(jnp.bfloat16)
        projs_chunks, ext_chunks = [], []
        for c0 in range(0, SEQ, QT):
            scb = (iota_cr + c0) == rank_row
            projs_chunks.append(jnp.dot(scb.astype(jnp.bfloat16), projb,
                                        preferred_element_type=jnp.float32))
            ext_chunks.append(jnp.dot(scb.astype(jnp.float32), ext,
                                      preferred_element_type=jnp.float32))
        projs = jnp.concatenate(projs_chunks, axis=0)
        exts = jnp.concatenate(ext_chunks, axis=0)
        ps_col = exts[:, 0:1]
        rc_col = exts[:, 1:2]
        ps_row = jnp.transpose(ps_col)
        rc_row = jnp.transpose(rc_col)

        qs_ref[...] = (jnp.dot(projs, w_q_ref[...],
                               preferred_element_type=jnp.float32)
                       + b_q_ref[...]) * SCALE
        ks = jnp.dot(projs, w_k_ref[...],
                     preferred_element_type=jnp.float32) + b_k_ref[...]
        vs = jnp.dot(projs, w_v_ref[...],
                     preferred_element_type=jnp.float32) + b_v_ref[...]
        ksp_ref[MARGIN:MARGIN + SEQ, :] = ks
        ksp_ref[0:MARGIN, :] = jnp.zeros((MARGIN, P), jnp.float32)
        ksp_ref[MARGIN + SEQ:, :] = jnp.zeros((MARGIN, P), jnp.float32)
        vsp_ref[MARGIN:MARGIN + SEQ, :] = vs
        vsp_ref[0:MARGIN, :] = jnp.zeros((MARGIN, P), jnp.float32)
        vsp_ref[MARGIN + SEQ:, :] = jnp.zeros((MARGIN, P), jnp.float32)
        psp_ref[:, MARGIN:MARGIN + SEQ] = ps_row
        psp_ref[:, 0:MARGIN] = jnp.full((1, MARGIN), -1.0, jnp.float32)
        psp_ref[:, MARGIN + SEQ:] = jnp.full((1, MARGIN), 2.0, jnp.float32)
        rcp_ref[:, MARGIN:MARGIN + SEQ] = rc_row
        rcp_ref[:, 0:MARGIN] = jnp.zeros((1, MARGIN), jnp.float32)
        rcp_ref[:, MARGIN + SEQ:] = jnp.zeros((1, MARGIN), jnp.float32)
        psc_ref[...] = ps_col
        psr_ref[...] = ps_row
        hir_ref[...] = jnp.concatenate(
            [ps_row[:, KW - 1:],
             jnp.full((1, KW - 1), 2.0, jnp.float32)], axis=1)
        acc_ref[...] = jnp.zeros((1, P), jnp.float32)

    r0 = pid * QT
    x = psc_ref[pl.ds(r0, QT), :]
    ws0 = jnp.maximum(2 * pid - 1, 0) * KW
    lowv = psr_ref[:, pl.ds(ws0, WS)]
    highv = hir_ref[:, pl.ds(ws0, WS)]
    cand = jnp.maximum(x - lowv, highv - x)
    tt = jnp.min(cand, axis=-1, keepdims=True)

    psb = psp_ref[:, pl.ds(r0, BW)]
    rcb = rcp_ref[:, pl.ds(r0, BW)]
    db = jnp.abs(x - psb)
    lt = db < tt
    rneed = (KW - jnp.sum(lt.astype(jnp.int32), axis=-1,
                          keepdims=True)).astype(jnp.float32)
    sel = (lt | ((db == tt) & (rcb < rneed))).astype(jnp.float32)

    outs = []
    for h in range(H):
        qh = qs_ref[pl.ds(r0, QT), h * HD:(h + 1) * HD]
        khb = ksp_ref[pl.ds(r0, BW), h * HD:(h + 1) * HD]
        vhb = vsp_ref[pl.ds(r0, BW), h * HD:(h + 1) * HD]
        s = jax.lax.dot_general(qh, khb, (((1,), (1,)), ((), ())),
                                preferred_element_type=jnp.float32)
        mh = (jnp.max(jnp.sqrt(jnp.sum(qh * qh, axis=-1, keepdims=True)))
              * jnp.max(jnp.sqrt(jnp.sum(khb * khb, axis=-1, keepdims=True))))
        e = jnp.exp(s - mh) * sel
        den = jnp.sum(e, axis=-1, keepdims=True)
        oh = jnp.dot(e, vhb, preferred_element_type=jnp.float32) / den
        outs.append(jnp.sum(oh, axis=0, keepdims=True))
    acc_ref[...] = acc_ref[...] + jnp.concatenate(outs, axis=-1)

    @pl.when(pid == NT - 1)
    def _epilogue():
        pooled = jnp.dot(acc_ref[...] / SEQ, w_out_ref[...],
                         preferred_element_type=jnp.float32) + b_out_ref[...]

        def opinion(wa, ba, g, b, wb, bo):
            hh = jnp.dot(pooled, wa[...],
                         preferred_element_type=jnp.float32) + ba[...]
            hh = _gelu(_ln(hh, g[...], b[...]))
            return jnp.dot(hh, wb[...],
                           preferred_element_type=jnp.float32) + bo[...]

        out0_ref[...] = opinion(wa0_ref, ba0_ref, g0_ref, bb0_ref,
                                wb0_ref, bo0_ref)
        out1_ref[...] = opinion(wa1_ref, ba1_ref, g1_ref, bb1_ref,
                                wb1_ref, bo1_ref)
        out2_ref[...] = opinion(wa2_ref, ba2_ref, g2_ref, bb2_ref,
                                wb2_ref, bo2_ref)


def kernel(sequence_features, params, shared_pentachora, shared_positions):
    x2d = sequence_features[0]
    pentT = jnp.transpose(shared_pentachora, (1, 2, 0))
    pos = shared_positions.reshape(1, NPENT)

    wqkv = params['W_qkv']
    bqkv = params['b_qkv']
    args = [
        x2d, pentT, pos,
        params['W_in'], params['b_in'].reshape(1, P),
        params['ln_in_g'].reshape(1, P), params['ln_in_b'].reshape(1, P),
        wqkv[:, :P], bqkv[:P].reshape(1, P),
        wqkv[:, P:2 * P], bqkv[P:2 * P].reshape(1, P),
        wqkv[:, 2 * P:], bqkv[2 * P:].reshape(1, P),
        params['W_out'], params['b_out'].reshape(1, P),
    ]
    for s in (64, 128, 256):
        args += [
            params[f'W_a_{s}'], params[f'b_a_{s}'].reshape(1, 2 * s),
            params[f'ln_g_{s}'].reshape(1, 2 * s),
            params[f'ln_b_{s}'].reshape(1, 2 * s),
            params[f'W_b_{s}'], params[f'b_b_{s}'].reshape(1, s),
        ]

    grid = (NT,)
    in_specs = [pl.BlockSpec(a.shape, lambda i, nd=a.ndim: (0,) * nd)
                for a in args]
    out_specs = [pl.BlockSpec((1, 64), lambda i: (0, 0)),
                 pl.BlockSpec((1, 128), lambda i: (0, 0)),
                 pl.BlockSpec((1, 256), lambda i: (0, 0))]
    o0, o1, o2 = pl.pallas_call(
        _megakernel,
        grid=grid,
        in_specs=in_specs,
        out_specs=out_specs,
        out_shape=[
            jax.ShapeDtypeStruct((1, 64), jnp.float32),
            jax.ShapeDtypeStruct((1, 128), jnp.float32),
            jax.ShapeDtypeStruct((1, 256), jnp.float32),
        ],
        scratch_shapes=[
            pltpu.VMEM((SEQ, P), jnp.float32),
            pltpu.VMEM((SEQ + 2 * MARGIN, P), jnp.float32),
            pltpu.VMEM((SEQ + 2 * MARGIN, P), jnp.float32),
            pltpu.VMEM((1, SEQ + 2 * MARGIN), jnp.float32),
            pltpu.VMEM((1, SEQ + 2 * MARGIN), jnp.float32),
            pltpu.VMEM((SEQ, 1), jnp.float32),
            pltpu.VMEM((1, SEQ), jnp.float32),
            pltpu.VMEM((1, SEQ), jnp.float32),
            pltpu.VMEM((1, P), jnp.float32),
        ],
    )(*args)
    return jnp.concatenate([o0, o1, o2], axis=-1)


# speedup vs baseline: 1.1174x; 1.1174x over previous
import math

import jax
import jax.numpy as jnp
from jax.experimental import pallas as pl
from jax.experimental.pallas import tpu as pltpu

SEQ = 2048
IN_DIM = 1024
P = 256
H = 8
HD = 32
NPENT = 1024
KW = 128
QT = 256
NT = SEQ // QT
MARGIN = 192
BW = QT + 2 * MARGIN
NW = SEQ - KW + 1
WS = 384
SCALE = 1.0 / math.sqrt(HD)


def _gelu(x):
    return 0.5 * x * (1.0 + jax.lax.erf(x * (1.0 / math.sqrt(2.0))))


def _ln(x, g, b):
    mu = jnp.mean(x, axis=-1, keepdims=True)
    xc = x - mu
    var = jnp.mean(xc * xc, axis=-1, keepdims=True)
    return xc / jnp.sqrt(var + 1e-5) * g + b


def _megakernel(x_ref, pentT_ref, pos_ref,
                w_in_ref, b_in_ref, g_in_ref, bt_in_ref,
                w_q_ref, b_q_ref, w_k_ref, b_k_ref, w_v_ref, b_v_ref,
                w_out_ref, b_out_ref,
                wa0_ref, ba0_ref, g0_ref, bb0_ref, wb0_ref, bo0_ref,
                wa1_ref, ba1_ref, g1_ref, bb1_ref, wb1_ref, bo1_ref,
                wa2_ref, ba2_ref, g2_ref, bb2_ref, wb2_ref, bo2_ref,
                out0_ref, out1_ref, out2_ref,
                qs_ref, ksp_ref, vsp_ref, psp_ref, rcp_ref,
                psc_ref, psr_ref, hir_ref, acc_ref):
    pid = pl.program_id(0)

    @pl.when(pid == 0)
    def _preamble():
        centT = (pentT_ref[0] + pentT_ref[1] + pentT_ref[2]
                 + pentT_ref[3] + pentT_ref[4]) / 5.0
        cn = jnp.sqrt(jnp.sum(centT * centT, axis=0, keepdims=True))
        centT = centT / jnp.maximum(cn, 1e-12)

        pr = jnp.dot(x_ref[...], w_in_ref[...],
                     preferred_element_type=jnp.float32) + b_in_ref[...]
        proj = _gelu(_ln(pr, g_in_ref[...], bt_in_ref[...]))

        n = jnp.sqrt(jnp.sum(proj * proj, axis=-1, keepdims=True))
        fn = proj / jnp.maximum(n, 1e-12)
        sims = jnp.dot(fn, centT, preferred_element_type=jnp.float32)
        rowmax = jnp.max(sims, axis=-1, keepdims=True)
        lane_a = jax.lax.broadcasted_iota(jnp.int32, (SEQ, NPENT), 1)
        aidx = jnp.min(jnp.where(sims == rowmax, lane_a, NPENT),
                       axis=-1, keepdims=True)
        onehot = (lane_a == aidx).astype(jnp.float32)
        pos_col = jnp.dot(onehot, jnp.transpose(pos_ref[...]),
                          preferred_element_type=jnp.float32)
        pos_row = jnp.transpose(pos_col)

        iota_cr = jax.lax.broadcasted_iota(jnp.int32, (QT, SEQ), 0)
        iota_cc = jax.lax.broadcasted_iota(jnp.int32, (QT, SEQ), 1)
        rcount_row = jnp.zeros((1, SEQ), jnp.int32)
        cntlt_row = jnp.zeros((1, SEQ), jnp.int32)
        for c0 in range(0, SEQ, QT):
            pc = pos_col[c0:c0 + QT, :]
            jlt = (iota_cr + c0) < iota_cc
            rcount_row = rcount_row + jnp.sum(
                ((pc == pos_row) & jlt).astype(jnp.int32),
                axis=0, keepdims=True)
            cntlt_row = cntlt_row + jnp.sum(
                (pc < pos_row).astype(jnp.int32), axis=0, keepdims=True)
        rank_row = cntlt_row + rcount_row

        ext = jnp.concatenate(
            [proj, pos_col, jnp.transpose(rcount_row).astype(jnp.float32)],
            axis=1)
        exts_chunks = []
        for c0 in range(0, SEQ, QT):
            scf = ((iota_cr + c0) == rank_row).astype(jnp.float32)
            exts_chunks.append(jnp.dot(scf, ext,
                                       preferred_element_type=jnp.float32))
        exts = jnp.concatenate(exts_chunks, axis=0)
        projs = exts[:, :P]
        ps_col = exts[:, P:P + 1]
        rc_col = exts[:, P + 1:]
        ps_row = jnp.transpose(ps_col)
        rc_row = jnp.transpose(rc_col)

        qs_ref[...] = (jnp.dot(projs, w_q_ref[...],
                               preferred_element_type=jnp.float32)
                       + b_q_ref[...]) * SCALE
        ks = jnp.dot(projs, w_k_ref[...],
                     preferred_element_type=jnp.float32) + b_k_ref[...]
        vs = jnp.dot(projs, w_v_ref[...],
                     preferred_element_type=jnp.float32) + b_v_ref[...]
        ksp_ref[MARGIN:MARGIN + SEQ, :] = ks
        ksp_ref[0:MARGIN, :] = jnp.zeros((MARGIN, P), jnp.float32)
        ksp_ref[MARGIN + SEQ:, :] = jnp.zeros((MARGIN, P), jnp.float32)
        vsp_ref[MARGIN:MARGIN + SEQ, :] = vs
        vsp_ref[0:MARGIN, :] = jnp.zeros((MARGIN, P), jnp.float32)
        vsp_ref[MARGIN + SEQ:, :] = jnp.zeros((MARGIN, P), jnp.float32)
        psp_ref[:, MARGIN:MARGIN + SEQ] = ps_row
        psp_ref[:, 0:MARGIN] = jnp.full((1, MARGIN), -1.0, jnp.float32)
        psp_ref[:, MARGIN + SEQ:] = jnp.full((1, MARGIN), 2.0, jnp.float32)
        rcp_ref[:, MARGIN:MARGIN + SEQ] = rc_row
        rcp_ref[:, 0:MARGIN] = jnp.zeros((1, MARGIN), jnp.float32)
        rcp_ref[:, MARGIN + SEQ:] = jnp.zeros((1, MARGIN), jnp.float32)
        psc_ref[...] = ps_col
        psr_ref[...] = ps_row
        hir_ref[...] = jnp.concatenate(
            [ps_row[:, KW - 1:],
             jnp.full((1, KW - 1), 2.0, jnp.float32)], axis=1)
        acc_ref[...] = jnp.zeros((1, P), jnp.float32)

    r0 = pid * QT
    x = psc_ref[pl.ds(r0, QT), :]
    ws0 = jnp.maximum(2 * pid - 1, 0) * KW
    lowv = psr_ref[:, pl.ds(ws0, WS)]
    highv = hir_ref[:, pl.ds(ws0, WS)]
    cand = jnp.maximum(x - lowv, highv - x)
    tt = jnp.min(cand, axis=-1, keepdims=True)

    psb = psp_ref[:, pl.ds(r0, BW)]
    rcb = rcp_ref[:, pl.ds(r0, BW)]
    db = jnp.abs(x - psb)
    lt = db < tt
    rneed = (KW - jnp.sum(lt.astype(jnp.int32), axis=-1,
                          keepdims=True)).astype(jnp.float32)
    sel = lt | ((db == tt) & (rcb < rneed))

    outs = []
    for h in range(H):
        qh = qs_ref[pl.ds(r0, QT), h * HD:(h + 1) * HD]
        khb = ksp_ref[pl.ds(r0, BW), h * HD:(h + 1) * HD]
        vhb = vsp_ref[pl.ds(r0, BW), h * HD:(h + 1) * HD]
        s = jax.lax.dot_general(qh, khb, (((1,), (1,)), ((), ())),
                                preferred_element_type=jnp.float32)
        s = jnp.where(sel, s, -1e30)
        m = jnp.max(s, axis=-1, keepdims=True)
        e = jnp.exp(s - m)
        den = jnp.sum(e, axis=-1, keepdims=True)
        oh = jnp.dot(e, vhb, preferred_element_type=jnp.float32) / den
        outs.append(jnp.sum(oh, axis=0, keepdims=True))
    acc_ref[...] = acc_ref[...] + jnp.concatenate(outs, axis=-1)

    @pl.when(pid == NT - 1)
    def _epilogue():
        pooled = jnp.dot(acc_ref[...] / SEQ, w_out_ref[...],
                         preferred_element_type=jnp.float32) + b_out_ref[...]

        def opinion(wa, ba, g, b, wb, bo):
            hh = jnp.dot(pooled, wa[...],
                         preferred_element_type=jnp.float32) + ba[...]
            hh = _gelu(_ln(hh, g[...], b[...]))
            return jnp.dot(hh, wb[...],
                           preferred_element_type=jnp.float32) + bo[...]

        out0_ref[...] = opinion(wa0_ref, ba0_ref, g0_ref, bb0_ref,
                                wb0_ref, bo0_ref)
        out1_ref[...] = opinion(wa1_ref, ba1_ref, g1_ref, bb1_ref,
                                wb1_ref, bo1_ref)
        out2_ref[...] = opinion(wa2_ref, ba2_ref, g2_ref, bb2_ref,
                                wb2_ref, bo2_ref)


def kernel(sequence_features, params, shared_pentachora, shared_positions):
    x2d = sequence_features[0]
    pentT = jnp.transpose(shared_pentachora, (1, 2, 0))
    pos = shared_positions.reshape(1, NPENT)

    wqkv = params['W_qkv']
    bqkv = params['b_qkv']
    args = [
        x2d, pentT, pos,
        params['W_in'], params['b_in'].reshape(1, P),
        params['ln_in_g'].reshape(1, P), params['ln_in_b'].reshape(1, P),
        wqkv[:, :P], bqkv[:P].reshape(1, P),
        wqkv[:, P:2 * P], bqkv[P:2 * P].reshape(1, P),
        wqkv[:, 2 * P:], bqkv[2 * P:].reshape(1, P),
        params['W_out'], params['b_out'].reshape(1, P),
    ]
    for s in (64, 128, 256):
        args += [
            params[f'W_a_{s}'], params[f'b_a_{s}'].reshape(1, 2 * s),
            params[f'ln_g_{s}'].reshape(1, 2 * s),
            params[f'ln_b_{s}'].reshape(1, 2 * s),
            params[f'W_b_{s}'], params[f'b_b_{s}'].reshape(1, s),
        ]

    grid = (NT,)
    in_specs = [pl.BlockSpec(a.shape, lambda i, nd=a.ndim: (0,) * nd)
                for a in args]
    out_specs = [pl.BlockSpec((1, 64), lambda i: (0, 0)),
                 pl.BlockSpec((1, 128), lambda i: (0, 0)),
                 pl.BlockSpec((1, 256), lambda i: (0, 0))]
    o0, o1, o2 = pl.pallas_call(
        _megakernel,
        grid=grid,
        in_specs=in_specs,
        out_specs=out_specs,
        out_shape=[
            jax.ShapeDtypeStruct((1, 64), jnp.float32),
            jax.ShapeDtypeStruct((1, 128), jnp.float32),
            jax.ShapeDtypeStruct((1, 256), jnp.float32),
        ],
        scratch_shapes=[
            pltpu.VMEM((SEQ, P), jnp.float32),
            pltpu.VMEM((SEQ + 2 * MARGIN, P), jnp.float32),
            pltpu.VMEM((SEQ + 2 * MARGIN, P), jnp.float32),
            pltpu.VMEM((1, SEQ + 2 * MARGIN), jnp.float32),
            pltpu.VMEM((1, SEQ + 2 * MARGIN), jnp.float32),
            pltpu.VMEM((SEQ, 1), jnp.float32),
            pltpu.VMEM((1, SEQ), jnp.float32),
            pltpu.VMEM((1, SEQ), jnp.float32),
            pltpu.VMEM((1, P), jnp.float32),
        ],
    )(*args)
    return jnp.concatenate([o0, o1, o2], axis=-1)
